# Initial kernel scaffold; baseline (speedup 1.0000x reference)
import jax
import jax.numpy as jnp
from jax.experimental import pallas as pl


def kernel(x, token_table, pos_table):
    raise NotImplementedError("write your pallas kernel here")


# speedup vs baseline: 2.2753x; 2.2753x over previous
import functools

import jax
import jax.numpy as jnp
from jax import lax
from jax.experimental import pallas as pl
from jax.experimental.pallas import tpu as pltpu
from jax.experimental.pallas import tpu_sc as plsc

_LANES = 16


def _gather_chunks(n):
  chunks = []
  off = 0
  while off < n:
    w = min(128, n - off)
    chunks.append((off, w))
    off += w
  return chunks


@functools.lru_cache(maxsize=None)
def _build(B, L, V, D, NC, NS):
  NW = NC * NS
  assert B % NW == 0, (B, NW)
  seq_per_w = B // NW
  assert D % _LANES == 0
  chunks = _gather_chunks(L)

  mesh = plsc.VectorSubcoreMesh(
      core_axis_name="c", subcore_axis_name="s",
      num_cores=NC, num_subcores=NS)

  @functools.partial(
      pl.kernel,
      out_type=jax.ShapeDtypeStruct((B * L, D), jnp.float32),
      mesh=mesh,
      scratch_types=[
          pltpu.VMEM((L, D), jnp.float32),
          pltpu.VMEM((L,), jnp.int32),
          pltpu.VMEM((L, D), jnp.float32),
          pltpu.SemaphoreType.DMA,
      ],
      compiler_params=pltpu.CompilerParams(use_tc_tiling_on_sc=False),
  )
  def k(x_hbm, tok_hbm, pos_hbm, out_hbm, pos_v, idx_v, rows_v, sem):
    wid = lax.axis_index("s") * NC + lax.axis_index("c")
    pltpu.sync_copy(pos_hbm, pos_v)

    def seq_body(i, carry):
      seq = wid * seq_per_w + i
      base = seq * L
      pltpu.sync_copy(x_hbm.at[seq], idx_v)
      cps = [
          pltpu.async_copy(
              tok_hbm.at[idx_v.at[pl.ds(off, w)]],
              rows_v.at[pl.ds(off, w)], sem)
          for off, w in chunks
      ]
      for cp in cps:
        cp.wait()

      def add_body(r, c2):
        for j in range(D // _LANES):
          sl = pl.ds(j * _LANES, _LANES)
          rows_v[r, sl] = rows_v[r, sl] + pos_v[r, sl]
        return c2

      lax.fori_loop(0, L, add_body, 0, unroll=2)
      pltpu.sync_copy(rows_v, out_hbm.at[pl.ds(base, L)])
      return carry

    lax.fori_loop(0, seq_per_w, seq_body, 0)

  return k


def kernel(x, token_table, pos_table):
  B, L = x.shape
  V, D = token_table.shape
  try:
    info = plsc.get_sparse_core_info()
    NC, NS = info.num_cores, info.num_subcores
  except Exception:
    NC, NS = 2, 16
  k = _build(B, L, V, D, NC, NS)
  out = k(x.astype(jnp.int32), token_table, pos_table)
  return out.reshape(B, L, D)


# speedup vs baseline: 2.7321x; 1.2008x over previous
import functools

import jax
import jax.numpy as jnp
from jax import lax
from jax.experimental import pallas as pl
from jax.experimental.pallas import tpu as pltpu
from jax.experimental.pallas import tpu_sc as plsc

_LANES = 16


def _gather_chunks(n):
  chunks = []
  off = 0
  while off < n:
    w = min(128, n - off)
    chunks.append((off, w))
    off += w
  return chunks


@functools.lru_cache(maxsize=None)
def _build(B, L, V, D, NC, NS):
  NW = NC * NS
  assert B % NW == 0 and (B // NW) % 2 == 0, (B, NW)
  spw = B // NW
  assert D % _LANES == 0
  chunks = _gather_chunks(L)

  mesh = plsc.VectorSubcoreMesh(
      core_axis_name="c", subcore_axis_name="s",
      num_cores=NC, num_subcores=NS)

  @functools.partial(
      pl.kernel,
      out_type=jax.ShapeDtypeStruct((B * L, D), jnp.float32),
      mesh=mesh,
      scratch_types=[
          pltpu.VMEM((L, D), jnp.float32),
          pltpu.VMEM((spw, L), jnp.int32),
          pltpu.VMEM((L, D), jnp.float32),
          pltpu.VMEM((L, D), jnp.float32),
          pltpu.SemaphoreType.DMA,
          pltpu.SemaphoreType.DMA,
          pltpu.SemaphoreType.DMA,
          pltpu.SemaphoreType.DMA,
      ],
      compiler_params=pltpu.CompilerParams(use_tc_tiling_on_sc=False),
  )
  def k(x_hbm, tok_hbm, pos_hbm, out_hbm, pos_v, idx_all,
        rows0, rows1, gsem0, gsem1, ssem0, ssem1):
    wid = lax.axis_index("s") * NC + lax.axis_index("c")
    base_seq = wid * spw
    pltpu.sync_copy(pos_hbm, pos_v)
    pltpu.sync_copy(x_hbm.at[pl.ds(base_seq, spw)], idx_all)
    bufs = ((rows0, gsem0, ssem0), (rows1, gsem1, ssem1))

    def issue_gather(i, rv, gsem):
      for off, w in chunks:
        pltpu.async_copy(
            tok_hbm.at[idx_all.at[i, pl.ds(off, w)]],
            rv.at[pl.ds(off, w)], gsem)

    def wait_gather(rv, gsem):
      pltpu.make_async_copy(out_hbm.at[pl.ds(0, L)], rv, gsem).wait()

    def wait_scatter(rv, ssem):
      pltpu.make_async_copy(rv, out_hbm.at[pl.ds(0, L)], ssem).wait()

    issue_gather(0, rows0, gsem0)

    def body(p, carry):
      for b in range(2):
        rv, gsem, ssem = bufs[b]
        nrv, ngsem, nssem = bufs[1 - b]
        i = p * 2 + b

        @pl.when(i + 1 < spw)
        def _issue_next():
          @pl.when(i >= 1)
          def _wait_prev_scatter():
            wait_scatter(nrv, nssem)
          issue_gather(i + 1, nrv, ngsem)

        wait_gather(rv, gsem)

        def add_body(r, c2):
          for j in range(D // _LANES):
            sl = pl.ds(j * _LANES, _LANES)
            rv[r, sl] = rv[r, sl] + pos_v[r, sl]
          return c2

        lax.fori_loop(0, L, add_body, 0, unroll=4)
        pltpu.async_copy(rv, out_hbm.at[pl.ds((base_seq + i) * L, L)], ssem)
      return carry

    lax.fori_loop(0, spw // 2, body, 0)
    wait_scatter(rows0, ssem0)
    wait_scatter(rows1, ssem1)

  return k


def kernel(x, token_table, pos_table):
  B, L = x.shape
  V, D = token_table.shape
  try:
    info = plsc.get_sparse_core_info()
    NC, NS = info.num_cores, info.num_subcores
  except Exception:
    NC, NS = 2, 16
  k = _build(B, L, V, D, NC, NS)
  out = k(x.astype(jnp.int32), token_table, pos_table)
  return out.reshape(B, L, D)
